# Initial kernel scaffold; baseline (speedup 1.0000x reference)
import functools

import jax
import jax.numpy as jnp
from jax import lax
from jax.experimental import pallas as pl
from jax.experimental.pallas import tpu as pltpu
from jax.experimental.pallas import tpu_sc as plsc

N = 10000
NP = 10240
E = 320000
D = 128
NTILES = 32
CHUNK = 128
NCHUNKS = 79
EPT = NCHUNKS * CHUNK
EPAD = NTILES * EPT
STRIPE = NP // 16


def _mm_body(x_ref, w_ref, as_ref, ad_ref, wh_ref, ee_ref):
    x = x_ref[...]
    wh = jnp.dot(x, w_ref[...], preferred_element_type=jnp.float32)
    wh_ref[...] = wh
    es = jnp.sum(wh * as_ref[...], axis=1)
    ed = jnp.sum(wh * ad_ref[...], axis=1)
    ee_ref[...] = jnp.concatenate([es[None, :], ed[None, :]], axis=0)


def _matmul_stage(x, W, a_src, a_dst):
    return pl.pallas_call(
        _mm_body,
        out_shape=[
            jax.ShapeDtypeStruct((NP, D), jnp.float32),
            jax.ShapeDtypeStruct((2, NP), jnp.float32),
        ],
    )(x, W, a_src.reshape(1, D), a_dst.reshape(1, D))


def _fin_mm_body(a0_ref, a1_ref, dc_ref, b_ref, w_ref, as_ref, ad_ref,
                 wh_ref, ee_ref):
    den = dc_ref[...] + 1e-16
    x = (a0_ref[...] + a1_ref[...]) / den + b_ref[...]
    x = jnp.maximum(x, 0.0)
    wh = jnp.dot(x, w_ref[...], preferred_element_type=jnp.float32)
    wh_ref[...] = wh
    es = jnp.sum(wh * as_ref[...], axis=1)
    ed = jnp.sum(wh * ad_ref[...], axis=1)
    ee_ref[...] = jnp.concatenate([es[None, :], ed[None, :]], axis=0)


def _finalize_matmul_stage(acc, den_col, b, W, a_src, a_dst):
    return pl.pallas_call(
        _fin_mm_body,
        out_shape=[
            jax.ShapeDtypeStruct((NP, D), jnp.float32),
            jax.ShapeDtypeStruct((2, NP), jnp.float32),
        ],
    )(acc[0], acc[1], den_col, b.reshape(1, D), W,
      a_src.reshape(1, D), a_dst.reshape(1, D))


def _fin_body(a0_ref, a1_ref, dc_ref, b_ref, o_ref):
    den = dc_ref[...] + 1e-16
    o_ref[...] = jnp.maximum((a0_ref[...] + a1_ref[...]) / den + b_ref[...],
                             0.0)


def _finalize_stage(acc, den_col, b):
    return pl.pallas_call(
        _fin_body,
        out_shape=jax.ShapeDtypeStruct((NP, D), jnp.float32),
    )(acc[0], acc[1], den_col, b.reshape(1, D))


def _sc_edge_stage(wh, ee, srcp, dstp):
    mesh = plsc.VectorSubcoreMesh(core_axis_name="c", subcore_axis_name="s")

    @functools.partial(
        pl.kernel,
        out_type=[
            jax.ShapeDtypeStruct((2, NP, D), jnp.float32),
            jax.ShapeDtypeStruct((2, NP), jnp.float32),
        ],
        mesh=mesh,
        scratch_types=[
            pltpu.VMEM((NP,), jnp.float32),
            pltpu.VMEM((NP,), jnp.float32),
            pltpu.VMEM((NCHUNKS, CHUNK), jnp.int32),
            pltpu.VMEM((NCHUNKS, CHUNK), jnp.int32),
            pltpu.VMEM((EPT,), jnp.float32),
            pltpu.VMEM((CHUNK, D), jnp.float32),
            pltpu.VMEM_SHARED((NP, D), jnp.float32),
            pltpu.VMEM_SHARED((NP,), jnp.float32),
        ],
    )
    def k(wh_hbm, ee_hbm, src_hbm, dst_hbm, acc_out, den_out,
          es_v, ed_v, src_v, dst_v, w_v, rows, acc_sh, den_sh):
        cid = lax.axis_index("c")
        sid = lax.axis_index("s")
        tid = cid * 16 + sid

        pltpu.sync_copy(ee_hbm.at[0], es_v)
        pltpu.sync_copy(ee_hbm.at[1], ed_v)
        pltpu.sync_copy(src_hbm.at[tid], src_v)
        pltpu.sync_copy(dst_hbm.at[tid], dst_v)

        zero16 = jnp.zeros((16,), jnp.float32)

        @pl.loop(0, CHUNK)
        def _(r):
            for c in range(D // 16):
                rows[r, pl.ds(c * 16, 16)] = zero16

        @pl.loop(0, STRIPE // CHUNK)
        def _(i):
            pltpu.sync_copy(rows, acc_sh.at[pl.ds(sid * STRIPE + i * CHUNK,
                                                  CHUNK)])

        @pl.loop(0, STRIPE // 16)
        def _(i):
            w_v[pl.ds(i * 16, 16)] = zero16

        pltpu.sync_copy(w_v.at[pl.ds(0, STRIPE)],
                        den_sh.at[pl.ds(sid * STRIPE, STRIPE)])

        plsc.subcore_barrier()

        @pl.loop(0, NCHUNKS)
        def _(j):
            for c in range(CHUNK // 16):
                sl = pl.ds(c * 16, 16)
                si = src_v[j, sl]
                di = dst_v[j, sl]
                s = plsc.load_gather(es_v, [si])
                d = plsc.load_gather(ed_v, [di])
                l = s + d
                l = jnp.maximum(l, 0.2 * l)
                w_v[pl.ds(j * CHUNK + c * 16, 16)] = jnp.exp(l)

        @pl.loop(0, NCHUNKS)
        def _(j):
            pltpu.sync_copy(w_v.at[pl.ds(j * CHUNK, CHUNK)],
                            den_sh.at[dst_v.at[j]], add=True)

        @pl.loop(0, NCHUNKS)
        def _(j):
            pltpu.sync_copy(wh_hbm.at[src_v.at[j]], rows)

            @pl.loop(0, CHUNK)
            def _(r):
                bidx = jnp.full((16,), j * CHUNK + r, jnp.int32)
                wvec = plsc.load_gather(w_v, [bidx])
                for c in range(D // 16):
                    sl = pl.ds(c * 16, 16)
                    rows[r, sl] = rows[r, sl] * wvec

            pltpu.sync_copy(rows, acc_sh.at[dst_v.at[j]], add=True)

        plsc.subcore_barrier()

        pltpu.sync_copy(acc_sh.at[pl.ds(sid * STRIPE, STRIPE)],
                        acc_out.at[cid].at[pl.ds(sid * STRIPE, STRIPE)])
        pltpu.sync_copy(den_sh.at[pl.ds(sid * STRIPE, STRIPE)],
                        den_out.at[cid].at[pl.ds(sid * STRIPE, STRIPE)])

    return k(wh, ee, srcp, dstp)


def kernel(h, edges, coords, W0, a_src0, a_dst0, b0, W1, a_src1, a_dst1, b1):
    h2 = h.reshape(N, D)
    c2 = coords.reshape(N, 3)
    x0 = jnp.zeros((NP, 136), jnp.float32)
    x0 = x0.at[:N, :D].set(h2).at[:N, D:D + 3].set(c2)
    W0f = jnp.zeros((136, D), jnp.float32).at[:D + 3].set(W0)

    src = edges[0].astype(jnp.int32)
    dst = edges[1].astype(jnp.int32)
    pad_ids = N + (jnp.arange(EPAD - E, dtype=jnp.int32) % (NP - N))
    srcp = jnp.concatenate([src, pad_ids]).reshape(NTILES, NCHUNKS, CHUNK)
    dstp = jnp.concatenate([dst, pad_ids]).reshape(NTILES, NCHUNKS, CHUNK)

    wh0, ee0 = _matmul_stage(x0, W0f, a_src0, a_dst0)
    acc0, den0 = _sc_edge_stage(wh0, ee0, srcp, dstp)
    den0c = (den0[0] + den0[1]).reshape(NP, 1)

    wh1, ee1 = _finalize_matmul_stage(acc0, den0c, b0, W1, a_src1, a_dst1)
    acc1, den1 = _sc_edge_stage(wh1, ee1, srcp, dstp)
    den1c = (den1[0] + den1[1]).reshape(NP, 1)

    y = _finalize_stage(acc1, den1c, b1)
    return y[:N].reshape(1, 1, N, D)


# speedup vs baseline: 9.3410x; 9.3410x over previous
import dataclasses
import functools

import jax
import jax.numpy as jnp
from jax import lax
from jax.experimental import pallas as pl
from jax.experimental.pallas import tpu as pltpu
from jax.experimental.pallas import tpu_sc as plsc

N = 10000
NP = 10240
E = 320000
D = 128
HD = 64
CHUNK = 128
NCHUNKS = 158
EPT = NCHUNKS * CHUNK
EPAD = 16 * EPT
STRIPE = NP // 16


def _mm_body(x_ref, w_ref, as_ref, ad_ref, wh_ref, es_ref, ed_ref):
    x = x_ref[...]
    wh = jnp.dot(x, w_ref[...], preferred_element_type=jnp.float32)
    wh_ref[...] = wh
    es_ref[...] = jnp.sum(wh * as_ref[...], axis=1)[None, :]
    ed_ref[...] = jnp.sum(wh * ad_ref[...], axis=1)[None, :]


def _matmul_stage(x, W, a_src, a_dst):
    return pl.pallas_call(
        _mm_body,
        out_shape=[
            jax.ShapeDtypeStruct((NP, D), jnp.float32),
            jax.ShapeDtypeStruct((1, NP), jnp.float32),
            jax.ShapeDtypeStruct((1, NP), jnp.float32),
        ],
    )(x, W, a_src.reshape(1, D), a_dst.reshape(1, D))


def _fin_mm_body(a0_ref, a1_ref, dc_ref, b_ref, w_ref, as_ref, ad_ref,
                 wh_ref, es_ref, ed_ref):
    inv = 1.0 / (dc_ref[...] + 1e-16)
    x = jnp.concatenate([a0_ref[...], a1_ref[...]], axis=1) * inv + b_ref[...]
    x = jnp.maximum(x, 0.0)
    wh = jnp.dot(x, w_ref[...], preferred_element_type=jnp.float32)
    wh_ref[...] = wh
    es_ref[...] = jnp.sum(wh * as_ref[...], axis=1)[None, :]
    ed_ref[...] = jnp.sum(wh * ad_ref[...], axis=1)[None, :]


def _finalize_matmul_stage(acc, den_col, b, W, a_src, a_dst):
    return pl.pallas_call(
        _fin_mm_body,
        out_shape=[
            jax.ShapeDtypeStruct((NP, D), jnp.float32),
            jax.ShapeDtypeStruct((1, NP), jnp.float32),
            jax.ShapeDtypeStruct((1, NP), jnp.float32),
        ],
    )(acc[0], acc[1], den_col, b.reshape(1, D), W,
      a_src.reshape(1, D), a_dst.reshape(1, D))


def _fin_body(a0_ref, a1_ref, dc_ref, b_ref, o_ref):
    inv = 1.0 / (dc_ref[...] + 1e-16)
    x = jnp.concatenate([a0_ref[...], a1_ref[...]], axis=1) * inv + b_ref[...]
    o_ref[...] = jnp.maximum(x, 0.0)


def _finalize_stage(acc, den_col, b):
    return pl.pallas_call(
        _fin_body,
        out_shape=jax.ShapeDtypeStruct((NP, D), jnp.float32),
    )(acc[0], acc[1], den_col, b.reshape(1, D))


def _sc_edge_stage(wh2, es, ed, srcp, dstp):
    mesh = plsc.VectorSubcoreMesh(core_axis_name="c", subcore_axis_name="s")
    cp = pltpu.CompilerParams(use_tc_tiling_on_sc=False)
    if "needs_layout_passes" in pltpu.CompilerParams.__dataclass_fields__:
        cp = dataclasses.replace(cp, needs_layout_passes=False)

    @functools.partial(
        pl.kernel,
        compiler_params=cp,
        out_type=[
            jax.ShapeDtypeStruct((2, NP, HD), jnp.float32),
            jax.ShapeDtypeStruct((2, NP), jnp.float32),
        ],
        mesh=mesh,
        scratch_types=[
            pltpu.VMEM((NP,), jnp.float32),
            pltpu.VMEM((NP,), jnp.float32),
            pltpu.VMEM((1, CHUNK), jnp.int32),
            pltpu.VMEM((1, CHUNK), jnp.int32),
            pltpu.VMEM((1, CHUNK), jnp.float32),
            pltpu.VMEM((CHUNK, D), jnp.float32),
            pltpu.VMEM((CHUNK, HD), jnp.float32),
            pltpu.VMEM_SHARED((NP, HD), jnp.float32),
            pltpu.VMEM_SHARED((NP,), jnp.float32),
        ],
    )
    def k(wh_hbm, es_hbm, ed_hbm, src_hbm, dst_hbm, acc_out, den_out,
          es_v, ed_v, src_c, dst_c, w_c, rows, half, acc_sh, den_sh):
        cid = lax.axis_index("c")
        sid = lax.axis_index("s")
        zero16 = jnp.zeros((16,), jnp.float32)

        @pl.loop(0, STRIPE // 16)
        def _(i):
            es_v[pl.ds(i * 16, 16)] = zero16

        pltpu.sync_copy(es_v.at[pl.ds(0, STRIPE)],
                        den_sh.at[pl.ds(sid * STRIPE, STRIPE)])

        @pl.loop(0, CHUNK)
        def _(r):
            for c in range(HD // 16):
                half[r, pl.ds(c * 16, 16)] = zero16

        @pl.loop(0, STRIPE // CHUNK)
        def _(i):
            pltpu.sync_copy(half, acc_sh.at[pl.ds(sid * STRIPE + i * CHUNK,
                                                  CHUNK)])

        pltpu.sync_copy(es_hbm, es_v)
        pltpu.sync_copy(ed_hbm, ed_v)

        plsc.subcore_barrier()

        fbase = cid * HD

        @pl.loop(0, NCHUNKS)
        def _(j):
            pltpu.sync_copy(src_hbm.at[sid].at[j], src_c.at[0])
            pltpu.sync_copy(dst_hbm.at[sid].at[j], dst_c.at[0])

            for c in range(CHUNK // 16):
                sl = pl.ds(c * 16, 16)
                si = src_c[0, sl]
                di = dst_c[0, sl]
                s = plsc.load_gather(es_v, [si])
                d = plsc.load_gather(ed_v, [di])
                l = s + d
                l = jnp.maximum(l, 0.2 * l)
                w_c[0, sl] = jnp.exp(l)

            pltpu.sync_copy(w_c.at[0], den_sh.at[dst_c.at[0]], add=True)

            pltpu.sync_copy(wh_hbm.at[src_c.at[0]], rows)

            @pl.loop(0, CHUNK)
            def _(r):
                bidx = jnp.full((16,), r, jnp.int32)
                zidx = jnp.zeros((16,), jnp.int32)
                wvec = plsc.load_gather(w_c, [zidx, bidx])
                for c in range(HD // 16):
                    half[r, pl.ds(c * 16, 16)] = (
                        rows[r, pl.ds(fbase + c * 16, 16)] * wvec)

            pltpu.sync_copy(half, acc_sh.at[dst_c.at[0]], add=True)

        plsc.subcore_barrier()

        @pl.loop(0, STRIPE // CHUNK)
        def _(i):
            base = sid * STRIPE + i * CHUNK
            pltpu.sync_copy(acc_sh.at[pl.ds(base, CHUNK)], half)
            pltpu.sync_copy(half, acc_out.at[cid].at[pl.ds(base, CHUNK)])
            pltpu.sync_copy(den_sh.at[pl.ds(base, CHUNK)], w_c.at[0])
            pltpu.sync_copy(w_c.at[0], den_out.at[cid].at[pl.ds(base, CHUNK)])

    return k(wh2, es, ed, srcp, dstp)


def kernel(h, edges, coords, W0, a_src0, a_dst0, b0, W1, a_src1, a_dst1, b1):
    h2 = h.reshape(N, D)
    c2 = coords.reshape(N, 3)
    x0 = jnp.zeros((NP, 136), jnp.float32)
    x0 = x0.at[:N, :D].set(h2).at[:N, D:D + 3].set(c2)
    W0f = jnp.zeros((136, D), jnp.float32).at[:D + 3].set(W0)

    src = edges[0].astype(jnp.int32)
    dst = edges[1].astype(jnp.int32)
    pad_ids = N + (jnp.arange(EPAD - E, dtype=jnp.int32) % (NP - N))
    srcp = jnp.concatenate([src, pad_ids]).reshape(16, NCHUNKS, CHUNK)
    dstp = jnp.concatenate([dst, pad_ids]).reshape(16, NCHUNKS, CHUNK)

    wh0, es0, ed0 = _matmul_stage(x0, W0f, a_src0, a_dst0)
    acc0, den0 = _sc_edge_stage(wh0, es0.reshape(NP), ed0.reshape(NP),
                                srcp, dstp)
    den0c = den0[0].reshape(NP, 1)

    wh1, es1, ed1 = _finalize_matmul_stage(acc0, den0c, b0, W1,
                                           a_src1, a_dst1)
    acc1, den1 = _sc_edge_stage(wh1, es1.reshape(NP), ed1.reshape(NP),
                                srcp, dstp)
    den1c = den1[0].reshape(NP, 1)

    y = _finalize_stage(acc1, den1c, b1)
    return y[:N].reshape(1, 1, N, D)
